# baseline (speedup 1.0000x reference)
import functools

import jax
import jax.numpy as jnp
from jax import lax
from jax.experimental import pallas as pl
from jax.experimental.pallas import tpu as pltpu
from jax.experimental.pallas import tpu_sc as plsc

N = 10000
D = 128
C = 64
E = 320000

NC = 2
NS = 16
NW = NC * NS
CHUNK = 128
NCHUNK = 80
NSUB = 2
SUB = CHUNK // NSUB
EPT = CHUNK * NCHUNK
EPAD = EPT * NW
SHIFT = 14
MASK = (1 << SHIFT) - 1
ACC_ROWS_PER_TILE = 632
NPAD = NS * ACC_ROWS_PER_TILE
ZCHUNKS = ACC_ROWS_PER_TILE // CHUNK
ZREM = ACC_ROWS_PER_TILE - ZCHUNKS * CHUNK
LANES = 16

_mesh = plsc.VectorSubcoreMesh(core_axis_name="c", subcore_axis_name="s")


def _zero_acc_slice(zeros2d, zbuf, acc, s):
    pltpu.sync_copy(zeros2d, zbuf)
    zbase = s * ACC_ROWS_PER_TILE
    for t in range(ZCHUNKS):
        pltpu.sync_copy(zbuf, acc.at[pl.ds(zbase + t * CHUNK, CHUNK)])
    pltpu.sync_copy(zbuf.at[pl.ds(0, ZREM)],
                    acc.at[pl.ds(zbase + ZCHUNKS * CHUNK, ZREM)])


PGCH = NCHUNK // 2


@functools.partial(
    pl.kernel,
    out_type=jax.ShapeDtypeStruct((NC, NPAD, D), jnp.float32),
    mesh=_mesh,
    scratch_types=[
        pltpu.VMEM((PGCH, CHUNK), jnp.int32),
        pltpu.VMEM((PGCH, CHUNK), jnp.int32),
        pltpu.VMEM((CHUNK, D), jnp.float32),
        pltpu.VMEM((CHUNK, D), jnp.float32),
        [pltpu.SemaphoreType.DMA] * NSUB,
        [pltpu.SemaphoreType.DMA] * NSUB,
        pltpu.VMEM_SHARED((NPAD, D), jnp.float32),
    ],
)
def _sc_prop(hws, src_t, dst_t, zeros2d, out, srcp, dstp, buf0, buf1, sem0, sem1, acc):
    c = lax.axis_index("c")
    s = lax.axis_index("s")
    wid = c * NS + s
    _zero_acc_slice(zeros2d, buf0, acc, s)
    plsc.subcore_barrier()

    def gissue(j, buf, sems):
        for u in range(NSUB):
            pltpu.async_copy(hws.at[srcp.at[j].at[pl.ds(u * SUB, SUB)]],
                             buf.at[pl.ds(u * SUB, SUB)], sems[u])

    def gwait(j, buf, sems):
        for u in range(NSUB):
            pltpu.make_async_copy(hws.at[srcp.at[j].at[pl.ds(u * SUB, SUB)]],
                                  buf.at[pl.ds(u * SUB, SUB)], sems[u]).wait()

    for pg in range(2):
        pltpu.sync_copy(src_t.at[wid].at[pl.ds(pg * PGCH, PGCH)], srcp)
        pltpu.sync_copy(dst_t.at[wid].at[pl.ds(pg * PGCH, PGCH)], dstp)
        gissue(0, buf0, sem0)
        gissue(1, buf1, sem1)

        def body(i, carry):
            j = 2 * i
            gwait(j, buf0, sem0)
            pltpu.sync_copy(buf0, acc.at[dstp.at[j]], add=True)
            gissue(j + 2, buf0, sem0)
            gwait(j + 1, buf1, sem1)
            pltpu.sync_copy(buf1, acc.at[dstp.at[j + 1]], add=True)
            gissue(j + 3, buf1, sem1)
            return carry

        lax.fori_loop(0, PGCH // 2 - 1, body, 0)
        gwait(PGCH - 2, buf0, sem0)
        pltpu.sync_copy(buf0, acc.at[dstp.at[PGCH - 2]], add=True)
        gwait(PGCH - 1, buf1, sem1)
        pltpu.sync_copy(buf1, acc.at[dstp.at[PGCH - 1]], add=True)

    plsc.subcore_barrier()
    zbase = s * ACC_ROWS_PER_TILE
    pltpu.sync_copy(acc.at[pl.ds(zbase, ACC_ROWS_PER_TILE)],
                    out.at[c].at[pl.ds(zbase, ACC_ROWS_PER_TILE)])


@functools.partial(
    pl.kernel,
    out_type=jax.ShapeDtypeStruct((NC, NPAD, D), jnp.float32),
    mesh=_mesh,
    scratch_types=[
        pltpu.VMEM((NCHUNK, CHUNK), jnp.int32),
        pltpu.VMEM((CHUNK, D), jnp.float32),
        pltpu.VMEM((CHUNK, D), jnp.float32),
        pltpu.VMEM_SHARED((NPAD, D), jnp.float32),
    ],
)
def _sc_degree(dst_t, zeros2d, ones2d, out, dst_v, zbuf, ones_v, acc):
    c = lax.axis_index("c")
    s = lax.axis_index("s")
    wid = c * NS + s
    pltpu.sync_copy(dst_t.at[wid], dst_v)
    pltpu.sync_copy(ones2d, ones_v)
    _zero_acc_slice(zeros2d, zbuf, acc, s)
    plsc.subcore_barrier()

    def body(j, carry):
        pltpu.sync_copy(ones_v, acc.at[dst_v.at[j]], add=True)
        return carry

    lax.fori_loop(0, NCHUNK, body, 0)
    plsc.subcore_barrier()
    zbase = s * ACC_ROWS_PER_TILE
    pltpu.sync_copy(acc.at[pl.ds(zbase, ACC_ROWS_PER_TILE)],
                    out.at[c].at[pl.ds(zbase, ACC_ROWS_PER_TILE)])


BLK = 1000
GRID = N // BLK


def _row(shape):
    return pl.BlockSpec(shape, lambda i: (i, 0))


def _full(shape):
    return pl.BlockSpec(shape, lambda i: tuple(0 for _ in shape))


def _tc_init_body(x_ref, deg_ref, win_ref, w0_ref, wc0_ref, bc_ref,
                  out_ref, hw_ref, hws_ref):
    xb = x_ref[...]
    dinv = lax.rsqrt(deg_ref[...])
    rep0 = jnp.dot(xb, win_ref[...], preferred_element_type=jnp.float32)
    out_ref[...] = jnp.dot(rep0, wc0_ref[...], preferred_element_type=jnp.float32) + bc_ref[...]
    hw = jnp.dot(xb, w0_ref[...], preferred_element_type=jnp.float32)
    hw_ref[...] = hw
    hws_ref[...] = hw * dinv


_tc_init = pl.pallas_call(
    _tc_init_body,
    grid=(GRID,),
    in_specs=[_row((BLK, D)), _row((BLK, 1)), _full((D, D)), _full((D, D)),
              _full((D, C)), _full((1, C))],
    out_specs=[_row((BLK, C)), _row((BLK, D)), _row((BLK, D))],
    out_shape=[jax.ShapeDtypeStruct((N, C), jnp.float32),
               jax.ShapeDtypeStruct((N, D), jnp.float32),
               jax.ShapeDtypeStruct((N, D), jnp.float32)],
)


def _tc_step_body(p_ref, hw_ref, deg_ref, b_ref, w_ref, wc_ref, oprev_ref,
                  out_ref, hwo_ref, hwso_ref):
    dinv = lax.rsqrt(deg_ref[...])
    agg = dinv * (p_ref[0] + p_ref[1]) + (dinv * dinv) * hw_ref[...]
    h = jnp.maximum(agg + b_ref[...], 0.0)
    out_ref[...] = oprev_ref[...] + jnp.dot(h, wc_ref[...], preferred_element_type=jnp.float32)
    hw = jnp.dot(h, w_ref[...], preferred_element_type=jnp.float32)
    hwo_ref[...] = hw
    hwso_ref[...] = hw * dinv


_tc_step = pl.pallas_call(
    _tc_step_body,
    grid=(GRID,),
    in_specs=[pl.BlockSpec((NC, BLK, D), lambda i: (0, i, 0)),
              _row((BLK, D)), _row((BLK, 1)), _full((1, D)), _full((D, D)),
              _full((D, C)), _row((BLK, C))],
    out_specs=[_row((BLK, C)), _row((BLK, D)), _row((BLK, D))],
    out_shape=[jax.ShapeDtypeStruct((N, C), jnp.float32),
               jax.ShapeDtypeStruct((N, D), jnp.float32),
               jax.ShapeDtypeStruct((N, D), jnp.float32)],
)


def _tc_final_body(p_ref, hw_ref, deg_ref, b_ref, wc_ref, oprev_ref, out_ref):
    dinv = lax.rsqrt(deg_ref[...])
    agg = dinv * (p_ref[0] + p_ref[1]) + (dinv * dinv) * hw_ref[...]
    h = jnp.maximum(agg + b_ref[...], 0.0)
    out_ref[...] = oprev_ref[...] + jnp.dot(h, wc_ref[...], preferred_element_type=jnp.float32)


_tc_final = pl.pallas_call(
    _tc_final_body,
    grid=(GRID,),
    in_specs=[pl.BlockSpec((NC, BLK, D), lambda i: (0, i, 0)),
              _row((BLK, D)), _row((BLK, 1)), _full((1, D)),
              _full((D, C)), _row((BLK, C))],
    out_specs=_row((BLK, C)),
    out_shape=jax.ShapeDtypeStruct((N, C), jnp.float32),
)


def kernel(x, edge_index, W_in, W0, b0, W1, b1, W2, b2, W3, b3, Wc, bc):
    pad = EPAD - E
    src = jnp.concatenate([edge_index[0], jnp.zeros((pad,), jnp.int32)])
    dst = jnp.concatenate([edge_index[1], jnp.full((pad,), N, jnp.int32)])
    src_t = src.reshape(NW, NCHUNK, CHUNK)
    dst_t = dst.reshape(NW, NCHUNK, CHUNK)
    zeros2d = jnp.zeros((CHUNK, D), jnp.float32)
    ones2d = jnp.ones((CHUNK, D), jnp.float32)

    degp = _sc_degree(dst_t, zeros2d, ones2d)
    deg = (degp[0, :N, 0] + degp[1, :N, 0] + 1.0).reshape(N, 1)

    out, hw, hws = _tc_init(x, deg, W_in, W0, Wc[0:D], bc.reshape(1, C))
    for i, (W_next, b) in enumerate(((W1, b0), (W2, b1), (W3, b2))):
        p = _sc_prop(hws, src_t, dst_t, zeros2d)
        out, hw, hws = _tc_step(p, hw, deg, b.reshape(1, D), W_next,
                                Wc[(i + 1) * D:(i + 2) * D], out)
    p = _sc_prop(hws, src_t, dst_t, zeros2d)
    out = _tc_final(p, hw, deg, b3.reshape(1, D), Wc[4 * D:5 * D], out)
    return out


# speedup vs baseline: 1.0033x; 1.0033x over previous
import functools

import jax
import jax.numpy as jnp
from jax import lax
from jax.experimental import pallas as pl
from jax.experimental.pallas import tpu as pltpu
from jax.experimental.pallas import tpu_sc as plsc

N = 10000
D = 128
C = 64
E = 320000

NC = 2
NS = 16
NW = NC * NS
CHUNK = 128
NCHUNK = 80
NSUB = 1
SUB = CHUNK // NSUB
EPT = CHUNK * NCHUNK
EPAD = EPT * NW
SHIFT = 14
MASK = (1 << SHIFT) - 1
ACC_ROWS_PER_TILE = 632
NPAD = NS * ACC_ROWS_PER_TILE
ZCHUNKS = ACC_ROWS_PER_TILE // CHUNK
ZREM = ACC_ROWS_PER_TILE - ZCHUNKS * CHUNK
LANES = 16

_mesh = plsc.VectorSubcoreMesh(core_axis_name="c", subcore_axis_name="s")


def _zero_acc_slice(zeros2d, zbuf, acc, s):
    pltpu.sync_copy(zeros2d, zbuf)
    zbase = s * ACC_ROWS_PER_TILE
    for t in range(ZCHUNKS):
        pltpu.sync_copy(zbuf, acc.at[pl.ds(zbase + t * CHUNK, CHUNK)])
    pltpu.sync_copy(zbuf.at[pl.ds(0, ZREM)],
                    acc.at[pl.ds(zbase + ZCHUNKS * CHUNK, ZREM)])


PGCH = NCHUNK // 2


@functools.partial(
    pl.kernel,
    out_type=jax.ShapeDtypeStruct((NC, NPAD, D), jnp.float32),
    mesh=_mesh,
    scratch_types=[
        pltpu.VMEM((PGCH, CHUNK), jnp.int32),
        pltpu.VMEM((PGCH, CHUNK), jnp.int32),
        pltpu.VMEM((CHUNK, D), jnp.float32),
        pltpu.VMEM((CHUNK, D), jnp.float32),
        [pltpu.SemaphoreType.DMA] * NSUB,
        [pltpu.SemaphoreType.DMA] * NSUB,
        pltpu.VMEM_SHARED((NPAD, D), jnp.float32),
    ],
)
def _sc_prop(hws, src_t, dst_t, zeros2d, out, srcp, dstp, buf0, buf1, sem0, sem1, acc):
    c = lax.axis_index("c")
    s = lax.axis_index("s")
    wid = c * NS + s
    _zero_acc_slice(zeros2d, buf0, acc, s)
    plsc.subcore_barrier()

    def gissue(j, buf, sems):
        for u in range(NSUB):
            pltpu.async_copy(hws.at[srcp.at[j].at[pl.ds(u * SUB, SUB)]],
                             buf.at[pl.ds(u * SUB, SUB)], sems[u])

    def gwait(j, buf, sems):
        for u in range(NSUB):
            pltpu.make_async_copy(hws.at[srcp.at[j].at[pl.ds(u * SUB, SUB)]],
                                  buf.at[pl.ds(u * SUB, SUB)], sems[u]).wait()

    for pg in range(2):
        pltpu.sync_copy(src_t.at[wid].at[pl.ds(pg * PGCH, PGCH)], srcp)
        pltpu.sync_copy(dst_t.at[wid].at[pl.ds(pg * PGCH, PGCH)], dstp)
        gissue(0, buf0, sem0)
        gissue(1, buf1, sem1)

        def body(i, carry):
            j = 2 * i
            gwait(j, buf0, sem0)
            pltpu.sync_copy(buf0, acc.at[dstp.at[j]], add=True)
            gissue(j + 2, buf0, sem0)
            gwait(j + 1, buf1, sem1)
            pltpu.sync_copy(buf1, acc.at[dstp.at[j + 1]], add=True)
            gissue(j + 3, buf1, sem1)
            return carry

        lax.fori_loop(0, PGCH // 2 - 1, body, 0)
        gwait(PGCH - 2, buf0, sem0)
        pltpu.sync_copy(buf0, acc.at[dstp.at[PGCH - 2]], add=True)
        gwait(PGCH - 1, buf1, sem1)
        pltpu.sync_copy(buf1, acc.at[dstp.at[PGCH - 1]], add=True)

    plsc.subcore_barrier()
    zbase = s * ACC_ROWS_PER_TILE
    pltpu.sync_copy(acc.at[pl.ds(zbase, ACC_ROWS_PER_TILE)],
                    out.at[c].at[pl.ds(zbase, ACC_ROWS_PER_TILE)])


@functools.partial(
    pl.kernel,
    out_type=jax.ShapeDtypeStruct((NC, NPAD, D), jnp.float32),
    mesh=_mesh,
    scratch_types=[
        pltpu.VMEM((NCHUNK, CHUNK), jnp.int32),
        pltpu.VMEM((CHUNK, D), jnp.float32),
        pltpu.VMEM((CHUNK, D), jnp.float32),
        pltpu.VMEM_SHARED((NPAD, D), jnp.float32),
    ],
)
def _sc_degree(dst_t, zeros2d, ones2d, out, dst_v, zbuf, ones_v, acc):
    c = lax.axis_index("c")
    s = lax.axis_index("s")
    wid = c * NS + s
    pltpu.sync_copy(dst_t.at[wid], dst_v)
    pltpu.sync_copy(ones2d, ones_v)
    _zero_acc_slice(zeros2d, zbuf, acc, s)
    plsc.subcore_barrier()

    def body(j, carry):
        pltpu.sync_copy(ones_v, acc.at[dst_v.at[j]], add=True)
        return carry

    lax.fori_loop(0, NCHUNK, body, 0)
    plsc.subcore_barrier()
    zbase = s * ACC_ROWS_PER_TILE
    pltpu.sync_copy(acc.at[pl.ds(zbase, ACC_ROWS_PER_TILE)],
                    out.at[c].at[pl.ds(zbase, ACC_ROWS_PER_TILE)])


BLK = 1000
GRID = N // BLK


def _row(shape):
    return pl.BlockSpec(shape, lambda i: (i, 0))


def _full(shape):
    return pl.BlockSpec(shape, lambda i: tuple(0 for _ in shape))


def _tc_init_body(x_ref, win_ref, w0_ref, wc0_ref, bc_ref, out_ref, hw_ref):
    xb = x_ref[...]
    rep0 = jnp.dot(xb, win_ref[...], preferred_element_type=jnp.float32)
    out_ref[...] = jnp.dot(rep0, wc0_ref[...], preferred_element_type=jnp.float32) + bc_ref[...]
    hw_ref[...] = jnp.dot(xb, w0_ref[...], preferred_element_type=jnp.float32)


_tc_init = pl.pallas_call(
    _tc_init_body,
    grid=(GRID,),
    in_specs=[_row((BLK, D)), _full((D, D)), _full((D, D)),
              _full((D, C)), _full((1, C))],
    out_specs=[_row((BLK, C)), _row((BLK, D))],
    out_shape=[jax.ShapeDtypeStruct((N, C), jnp.float32),
               jax.ShapeDtypeStruct((N, D), jnp.float32)],
)


def _tc_scale_body(hw_ref, deg_ref, hws_ref):
    hws_ref[...] = hw_ref[...] * lax.rsqrt(deg_ref[...])


_tc_scale = pl.pallas_call(
    _tc_scale_body,
    grid=(GRID,),
    in_specs=[_row((BLK, D)), _row((BLK, 1))],
    out_specs=_row((BLK, D)),
    out_shape=jax.ShapeDtypeStruct((N, D), jnp.float32),
)


def _tc_step_body(p_ref, hw_ref, deg_ref, b_ref, w_ref, wc_ref, oprev_ref,
                  out_ref, hwo_ref, hwso_ref):
    dinv = lax.rsqrt(deg_ref[...])
    agg = dinv * (p_ref[0] + p_ref[1]) + (dinv * dinv) * hw_ref[...]
    h = jnp.maximum(agg + b_ref[...], 0.0)
    out_ref[...] = oprev_ref[...] + jnp.dot(h, wc_ref[...], preferred_element_type=jnp.float32)
    hw = jnp.dot(h, w_ref[...], preferred_element_type=jnp.float32)
    hwo_ref[...] = hw
    hwso_ref[...] = hw * dinv


_tc_step = pl.pallas_call(
    _tc_step_body,
    grid=(GRID,),
    in_specs=[pl.BlockSpec((NC, BLK, D), lambda i: (0, i, 0)),
              _row((BLK, D)), _row((BLK, 1)), _full((1, D)), _full((D, D)),
              _full((D, C)), _row((BLK, C))],
    out_specs=[_row((BLK, C)), _row((BLK, D)), _row((BLK, D))],
    out_shape=[jax.ShapeDtypeStruct((N, C), jnp.float32),
               jax.ShapeDtypeStruct((N, D), jnp.float32),
               jax.ShapeDtypeStruct((N, D), jnp.float32)],
)


def _tc_final_body(p_ref, hw_ref, deg_ref, b_ref, wc_ref, oprev_ref, out_ref):
    dinv = lax.rsqrt(deg_ref[...])
    agg = dinv * (p_ref[0] + p_ref[1]) + (dinv * dinv) * hw_ref[...]
    h = jnp.maximum(agg + b_ref[...], 0.0)
    out_ref[...] = oprev_ref[...] + jnp.dot(h, wc_ref[...], preferred_element_type=jnp.float32)


_tc_final = pl.pallas_call(
    _tc_final_body,
    grid=(GRID,),
    in_specs=[pl.BlockSpec((NC, BLK, D), lambda i: (0, i, 0)),
              _row((BLK, D)), _row((BLK, 1)), _full((1, D)),
              _full((D, C)), _row((BLK, C))],
    out_specs=_row((BLK, C)),
    out_shape=jax.ShapeDtypeStruct((N, C), jnp.float32),
)


def kernel(x, edge_index, W_in, W0, b0, W1, b1, W2, b2, W3, b3, Wc, bc):
    pad = EPAD - E
    src = jnp.concatenate([edge_index[0], jnp.zeros((pad,), jnp.int32)])
    dst = jnp.concatenate([edge_index[1], jnp.full((pad,), N, jnp.int32)])
    src_t = src.reshape(NW, NCHUNK, CHUNK)
    dst_t = dst.reshape(NW, NCHUNK, CHUNK)
    zeros2d = jnp.zeros((CHUNK, D), jnp.float32)
    ones2d = jnp.ones((CHUNK, D), jnp.float32)

    degp = _sc_degree(dst_t, zeros2d, ones2d)
    deg = (degp[0, :N, 0] + degp[1, :N, 0] + 1.0).reshape(N, 1)

    out, hw = _tc_init(x, W_in, W0, Wc[0:D], bc.reshape(1, C))
    hws = _tc_scale(hw, deg)
    for i, (W_next, b) in enumerate(((W1, b0), (W2, b1), (W3, b2))):
        p = _sc_prop(hws, src_t, dst_t, zeros2d)
        out, hw, hws = _tc_step(p, hw, deg, b.reshape(1, D), W_next,
                                Wc[(i + 1) * D:(i + 2) * D], out)
    p = _sc_prop(hws, src_t, dst_t, zeros2d)
    out = _tc_final(p, hw, deg, b3.reshape(1, D), Wc[4 * D:5 * D], out)
    return out
